# baseline (speedup 1.0000x reference)
import functools

import jax
import jax.numpy as jnp
from jax import lax
from jax.experimental import pallas as pl
from jax.experimental.pallas import tpu as pltpu
from jax.experimental.pallas import tpu_sc as plsc

_K = 20
_NW = 32


def _make_sc_gather(D, B, dtype):
    assert B % (8 * _NW) == 0 and D % 8 == 0
    b_per_w = B // _NW
    ch = 128
    while ch * D * 4 > 260 * 1024 and ch > 8:
        ch //= 2
    while b_per_w % ch:
        ch //= 2
    n_ch = b_per_w // ch
    mesh = plsc.VectorSubcoreMesh(core_axis_name="c", subcore_axis_name="s")

    @functools.partial(
        pl.kernel,
        mesh=mesh,
        compiler_params=pltpu.CompilerParams(use_tc_tiling_on_sc=False),
        out_type=jax.ShapeDtypeStruct((B, D), dtype),
        scratch_types=[
            pltpu.VMEM((b_per_w,), jnp.int32),
            pltpu.VMEM((ch, D), dtype),
            pltpu.SemaphoreType.DMA,
        ],
    )
    def k(table_hbm, idx_hbm, out_hbm, idx_v, rows_v, sem):
        wid = lax.axis_index("s") * 2 + lax.axis_index("c")
        base = pl.multiple_of(wid * b_per_w, 8)
        pltpu.sync_copy(idx_hbm.at[pl.ds(base, b_per_w)], idx_v)

        def body(c, carry):
            off = pl.multiple_of(c * ch, 8)
            pltpu.async_copy(
                table_hbm.at[idx_v.at[pl.ds(off, ch)]], rows_v, sem
            ).wait()
            pltpu.sync_copy(rows_v, out_hbm.at[pl.ds(base + off, ch)])
            return carry

        lax.fori_loop(0, n_ch, body, 0)

    return k


def _sc_gather(table, idx):
    n, d = table.shape
    (b,) = idx.shape
    return _make_sc_gather(d, b, table.dtype)(table, idx)


def _global_idx(ni, vcount):
    bs = ni.shape[0]
    off = (jnp.arange(bs, dtype=jnp.int32) * vcount)[:, None, None]
    return (ni.astype(jnp.int32) + off).reshape(-1)


def _gather_nbr(t, idx):
    bs, v, c = t.shape
    _, r, n = idx.shape
    g = _sc_gather(t.reshape(bs * v, c), _global_idx(idx, v))
    return g.reshape(bs, r, n, c)


def _gather_vert(vtx, idx):
    bs, v, _ = vtx.shape
    vp = jnp.pad(vtx, ((0, 0), (0, 0), (0, 13)))
    g = _sc_gather(vp.reshape(bs * v, 16), _global_idx(idx, v))
    return g.reshape(bs, v, idx.shape[2], 16)[..., :3]


def _normalize(x, axis):
    norm = jnp.linalg.norm(x, axis=axis, keepdims=True)
    return x / jnp.maximum(norm, 1e-12)


def _knn(v, k):
    inner = jnp.einsum('bvd,bwd->bvw', v, v)
    quad = jnp.sum(v ** 2, axis=2)
    d = -2.0 * inner + quad[:, None, :] + quad[:, :, None]
    _, idx = jax.lax.top_k(-d, k + 1)
    return idx[:, :, 1:]


def _ndn(v, ni):
    nb = _gather_vert(v, ni)
    return _normalize(nb - v[:, :, None, :], -1)


def _conv_surface(ndn, dirs):
    sdn = _normalize(dirs, 0)
    theta = jax.nn.relu(ndn @ sdn)
    return jnp.max(theta, axis=2)


def _conv_layer(ni, ndn, fm, w, b, dirs, oc):
    sdn = _normalize(dirs, 0)
    theta = jax.nn.relu(ndn @ sdn)
    fo = fm @ w + b
    center = fo[:, :, :oc]
    support = _gather_nbr(fo[:, :, oc:], ni)
    act = jnp.max(theta * support, axis=2)
    return center + act


def _pool(v, fm, ni, rate):
    bs, vn, _ = v.shape
    samp = jnp.arange(vn // rate) * rate
    nf = _gather_nbr(fm, ni[:, samp, :])
    pooled = jnp.max(nf, axis=2)
    return v[:, samp, :], pooled


def kernel(vertices, dirs0, w1, b1, dirs1, w2, b2, dirs2, w3, b3, dirs3, w4, b4, dirs4):
    bs, _, vn, _ = vertices.shape
    v = vertices.reshape(bs, vn, 3)
    ni = _knn(v, _K)
    ndn = _ndn(v, ni)
    fm0 = jax.nn.relu(_conv_surface(ndn, dirs0))
    fm1 = jax.nn.relu(_conv_layer(ni, ndn, fm0, w1, b1, dirs1, 64))
    v, fm1 = _pool(v, fm1, ni, 4)
    ni = _knn(v, _K)
    ndn = _ndn(v, ni)
    fm2 = jax.nn.relu(_conv_layer(ni, ndn, fm1, w2, b2, dirs2, 128))
    fm3 = jax.nn.relu(_conv_layer(ni, ndn, fm2, w3, b3, dirs3, 256))
    v, fm3 = _pool(v, fm3, ni, 4)
    ni = _knn(v, _K)
    ndn = _ndn(v, ni)
    fm4 = _conv_layer(ni, ndn, fm3, w4, b4, dirs4, 1024)
    fm4 = jnp.transpose(fm4, (0, 2, 1))[..., None]
    return fm4


# speedup vs baseline: 2.1887x; 2.1887x over previous
import functools

import jax
import jax.numpy as jnp
from jax import lax
from jax.experimental import pallas as pl
from jax.experimental.pallas import tpu as pltpu
from jax.experimental.pallas import tpu_sc as plsc

_K = 20
_NW = 32


def _make_sc_gather(D, B, dtype):
    assert B % (8 * _NW) == 0 and D % 8 == 0
    b_per_w = B // _NW
    ch = 128
    while ch * D * 4 > 260 * 1024 and ch > 8:
        ch //= 2
    while b_per_w % ch:
        ch //= 2
    n_ch = b_per_w // ch
    mesh = plsc.VectorSubcoreMesh(core_axis_name="c", subcore_axis_name="s")

    @functools.partial(
        pl.kernel,
        mesh=mesh,
        compiler_params=pltpu.CompilerParams(use_tc_tiling_on_sc=False),
        out_type=jax.ShapeDtypeStruct((B, D), dtype),
        scratch_types=[
            pltpu.VMEM((b_per_w,), jnp.int32),
            pltpu.VMEM((ch, D), dtype),
            pltpu.SemaphoreType.DMA,
        ],
    )
    def k(table_hbm, idx_hbm, out_hbm, idx_v, rows_v, sem):
        wid = lax.axis_index("s") * 2 + lax.axis_index("c")
        base = pl.multiple_of(wid * b_per_w, 8)
        pltpu.sync_copy(idx_hbm.at[pl.ds(base, b_per_w)], idx_v)

        def body(c, carry):
            off = pl.multiple_of(c * ch, 8)
            pltpu.async_copy(
                table_hbm.at[idx_v.at[pl.ds(off, ch)]], rows_v, sem
            ).wait()
            pltpu.sync_copy(rows_v, out_hbm.at[pl.ds(base + off, ch)])
            return carry

        lax.fori_loop(0, n_ch, body, 0)

    return k


def _sc_gather(table, idx):
    n, d = table.shape
    (b,) = idx.shape
    return _make_sc_gather(d, b, table.dtype)(table, idx)


def _global_idx(ni, vcount):
    bs = ni.shape[0]
    off = (jnp.arange(bs, dtype=jnp.int32) * vcount)[:, None, None]
    return (ni.astype(jnp.int32) + off).reshape(-1)


def _gather_nbr(t, idx):
    bs, v, c = t.shape
    _, r, n = idx.shape
    g = _sc_gather(t.reshape(bs * v, c), _global_idx(idx, v))
    return g.reshape(bs, r, n, c)


def _gather_vert(vtx, idx):
    bs, v, _ = vtx.shape
    vp = jnp.pad(vtx, ((0, 0), (0, 0), (0, 13)))
    g = _sc_gather(vp.reshape(bs * v, 16), _global_idx(idx, v))
    return g.reshape(bs, v, idx.shape[2], 16)[..., :3]


_CAP = 64


def _make_knn_prep(bs, V, RB):
    nslots = V // 128

    def body(vt_ref, vtf_ref, d_ref, tau_ref):
        vrow = vt_ref[0]
        vall = vtf_ref[0]
        inner = lax.dot_general(vrow, vall, (((0,), (0,)), ((), ())))
        q = jnp.sum(vall * vall, axis=0)
        qr = jnp.sum(vrow * vrow, axis=0)
        d = qr[:, None] + q[None, :] - 2.0 * inner
        rblk = pl.program_id(1)
        rowg = rblk * RB + lax.broadcasted_iota(jnp.int32, (RB, V), 0)
        colg = lax.broadcasted_iota(jnp.int32, (RB, V), 1)
        d = jnp.where(rowg == colg, jnp.inf, d)
        d_ref[0] = d
        fold = jnp.full((RB, 128), jnp.inf, jnp.float32)
        for s in range(nslots):
            fold = jnp.minimum(fold, d[:, s * 128:(s + 1) * 128])
        lane = lax.broadcasted_iota(jnp.int32, (RB, 128), 1)
        tau = None
        for _ in range(20):
            m = jnp.min(fold, axis=1)
            cand = jnp.where(fold == m[:, None], lane, 1 << 30)
            aml = jnp.min(cand, axis=1)
            fold = jnp.where(lane == aml[:, None], jnp.inf, fold)
            tau = m
        tau_ref[0] = jnp.broadcast_to(tau[:, None], (RB, 16))

    nrb = V // RB
    return pl.pallas_call(
        body,
        grid=(bs, nrb),
        in_specs=[
            pl.BlockSpec((1, 3, RB), lambda b, r: (b, 0, r)),
            pl.BlockSpec((1, 3, V), lambda b, r: (b, 0, 0)),
        ],
        out_specs=[
            pl.BlockSpec((1, RB, V), lambda b, r: (b, r, 0)),
            pl.BlockSpec((1, RB, 16), lambda b, r: (b * nrb + r, 0, 0)),
        ],
        out_shape=[
            jax.ShapeDtypeStruct((bs, V, V), jnp.float32),
            jax.ShapeDtypeStruct((bs * nrb, RB, 16), jnp.float32),
        ],
    )


def _make_sc_filter(R, V):
    rpt = R // _NW
    assert rpt % 16 == 0
    n_ch = V // 16
    mesh = plsc.VectorSubcoreMesh(core_axis_name="c", subcore_axis_name="s")
    inf16 = float('inf')

    @functools.partial(
        pl.kernel,
        mesh=mesh,
        compiler_params=pltpu.CompilerParams(
            use_tc_tiling_on_sc=False, needs_layout_passes=False
        ),
        out_type=[
            jax.ShapeDtypeStruct((R * _CAP,), jnp.float32),
            jax.ShapeDtypeStruct((R * _CAP,), jnp.int32),
        ],
        scratch_types=[
            pltpu.VMEM((1, V), jnp.float32),
            pltpu.VMEM((1, V), jnp.float32),
            pltpu.VMEM((rpt * 16,), jnp.float32),
            pltpu.VMEM((16 * _CAP,), jnp.float32),
            pltpu.VMEM((16 * _CAP,), jnp.int32),
            pltpu.SemaphoreType.DMA,
            pltpu.SemaphoreType.DMA,
        ],
    )
    def k(d_hbm, tau_hbm, val_hbm, idx_hbm, db0, db1, taub, vbuf, ibuf, s0, s1):
        wid = lax.axis_index("s") * 2 + lax.axis_index("c")
        base = pl.multiple_of(wid * rpt, 8)
        pltpu.sync_copy(tau_hbm.at[pl.ds(base * 16, rpt * 16)], taub)

        def prefill():
            big = jnp.full((16,), jnp.inf, jnp.float32)
            for j in range(_CAP):
                vbuf[pl.ds(j * 16, 16)] = big

        def process(dref, r):
            gi = lax.rem(r, 16)
            li = lax.iota(jnp.int32, 16)
            tauv = taub[pl.ds(pl.multiple_of(r * 16, 8), 16)]
            gb = jnp.full((16,), gi * _CAP, jnp.int32)

            def chunk(c, n):
                dv = dref[0, pl.ds(pl.multiple_of(c * 16, 8), 16)]
                mask = dv <= tauv
                cum = plsc.cumsum(jnp.where(mask, 1, 0))
                pos = jnp.clip(n + cum - 1, 0, _CAP - 1)
                colv = li + c * 16
                plsc.store_scatter(vbuf, [gb + pos], dv, mask=mask)
                plsc.store_scatter(ibuf, [gb + pos], colv, mask=mask)
                return n + plsc.all_reduce_population_count(mask)

            lax.fori_loop(0, n_ch, chunk, jnp.zeros((16,), jnp.int32))

            @pl.when(gi == 15)
            def _():
                off = pl.multiple_of((base + r - 15) * _CAP, 8)
                pltpu.sync_copy(vbuf, val_hbm.at[pl.ds(off, 16 * _CAP)])
                pltpu.sync_copy(ibuf, idx_hbm.at[pl.ds(off, 16 * _CAP)])
                prefill()

        prefill()
        pltpu.async_copy(d_hbm.at[pl.ds(base, 1)], db0, s0)

        def outer(i, carry):
            r = 2 * i
            pltpu.async_copy(d_hbm.at[pl.ds(base + r + 1, 1)], db1, s1)
            pltpu.make_async_copy(d_hbm.at[pl.ds(base + r, 1)], db0, s0).wait()
            process(db0, r)

            @pl.when(i + 1 < rpt // 2)
            def _():
                pltpu.async_copy(d_hbm.at[pl.ds(base + r + 2, 1)], db0, s0)

            pltpu.make_async_copy(
                d_hbm.at[pl.ds(base + r + 1, 1)], db1, s1
            ).wait()
            process(db1, r + 1)
            return carry

        lax.fori_loop(0, rpt // 2, outer, 0)

    return k


def _make_knn_extract(R, RB):
    def body(val_ref, idx_ref, out_ref):
        v = val_ref[...]
        ii = idx_ref[...]
        lane = lax.broadcasted_iota(jnp.int32, (RB, _CAP), 1)
        for kk in range(20):
            m = jnp.min(v, axis=1)
            cand = jnp.where(v == m[:, None], lane, 1 << 30)
            aml = jnp.min(cand, axis=1)
            oh = lane == aml[:, None]
            nik = jnp.sum(jnp.where(oh, ii, 0), axis=1)
            out_ref[:, kk] = nik
            v = jnp.where(oh, jnp.inf, v)

    return pl.pallas_call(
        body,
        grid=(R // RB,),
        in_specs=[
            pl.BlockSpec((RB, _CAP), lambda r: (r, 0)),
            pl.BlockSpec((RB, _CAP), lambda r: (r, 0)),
        ],
        out_specs=pl.BlockSpec((RB, 32), lambda r: (r, 0)),
        out_shape=jax.ShapeDtypeStruct((R, 32), jnp.int32),
    )


def _knn_pallas(v):
    bs, V, _ = v.shape
    R = bs * V
    vt = jnp.transpose(v, (0, 2, 1))
    d, tau = _make_knn_prep(bs, V, min(256, V))(vt, vt)
    cval, cidx = _make_sc_filter(R, V)(
        d.reshape(R, V), tau.reshape(R * 16)
    )
    ni = _make_knn_extract(R, 512)(
        cval.reshape(R, _CAP), cidx.reshape(R, _CAP)
    )
    return ni.reshape(bs, V, 32)[:, :, :20]


def _normalize(x, axis):
    norm = jnp.linalg.norm(x, axis=axis, keepdims=True)
    return x / jnp.maximum(norm, 1e-12)


def _knn(v, k):
    del k
    return _knn_pallas(v)


def _ndn(v, ni):
    nb = _gather_vert(v, ni)
    return _normalize(nb - v[:, :, None, :], -1)


def _conv_surface(ndn, dirs):
    sdn = _normalize(dirs, 0)
    theta = jax.nn.relu(ndn @ sdn)
    return jnp.max(theta, axis=2)


def _conv_layer(ni, ndn, fm, w, b, dirs, oc):
    sdn = _normalize(dirs, 0)
    theta = jax.nn.relu(ndn @ sdn)
    fo = fm @ w + b
    center = fo[:, :, :oc]
    support = _gather_nbr(fo[:, :, oc:], ni)
    act = jnp.max(theta * support, axis=2)
    return center + act


def _pool(v, fm, ni, rate):
    bs, vn, _ = v.shape
    samp = jnp.arange(vn // rate) * rate
    nf = _gather_nbr(fm, ni[:, samp, :])
    pooled = jnp.max(nf, axis=2)
    return v[:, samp, :], pooled


def kernel(vertices, dirs0, w1, b1, dirs1, w2, b2, dirs2, w3, b3, dirs3, w4, b4, dirs4):
    bs, _, vn, _ = vertices.shape
    v = vertices.reshape(bs, vn, 3)
    ni = _knn(v, _K)
    ndn = _ndn(v, ni)
    fm0 = jax.nn.relu(_conv_surface(ndn, dirs0))
    fm1 = jax.nn.relu(_conv_layer(ni, ndn, fm0, w1, b1, dirs1, 64))
    v, fm1 = _pool(v, fm1, ni, 4)
    ni = _knn(v, _K)
    ndn = _ndn(v, ni)
    fm2 = jax.nn.relu(_conv_layer(ni, ndn, fm1, w2, b2, dirs2, 128))
    fm3 = jax.nn.relu(_conv_layer(ni, ndn, fm2, w3, b3, dirs3, 256))
    v, fm3 = _pool(v, fm3, ni, 4)
    ni = _knn(v, _K)
    ndn = _ndn(v, ni)
    fm4 = _conv_layer(ni, ndn, fm3, w4, b4, dirs4, 1024)
    fm4 = jnp.transpose(fm4, (0, 2, 1))[..., None]
    return fm4


# speedup vs baseline: 2.2402x; 1.0235x over previous
import functools

import jax
import jax.numpy as jnp
from jax import lax
from jax.experimental import pallas as pl
from jax.experimental.pallas import tpu as pltpu
from jax.experimental.pallas import tpu_sc as plsc

_K = 20
_NW = 32


def _make_sc_gather(D, B, dtype):
    assert B % (8 * _NW) == 0 and D % 8 == 0
    b_per_w = B // _NW
    ch = 128
    while ch * D * 4 > 130 * 1024 and ch > 8:
        ch //= 2
    while b_per_w % (2 * ch):
        ch //= 2
    n_ch = b_per_w // ch
    assert n_ch % 2 == 0
    mesh = plsc.VectorSubcoreMesh(core_axis_name="c", subcore_axis_name="s")

    @functools.partial(
        pl.kernel,
        mesh=mesh,
        compiler_params=pltpu.CompilerParams(use_tc_tiling_on_sc=False),
        out_type=jax.ShapeDtypeStruct((B, D), dtype),
        scratch_types=[
            pltpu.VMEM((b_per_w,), jnp.int32),
            pltpu.VMEM((ch, D), dtype),
            pltpu.VMEM((ch, D), dtype),
            pltpu.SemaphoreType.DMA,
            pltpu.SemaphoreType.DMA,
        ],
    )
    def k(table_hbm, idx_hbm, out_hbm, idx_v, rv0, rv1, s0, s1):
        wid = lax.axis_index("s") * 2 + lax.axis_index("c")
        base = pl.multiple_of(wid * b_per_w, 8)
        pltpu.sync_copy(idx_hbm.at[pl.ds(base, b_per_w)], idx_v)
        pltpu.async_copy(table_hbm.at[idx_v.at[pl.ds(0, ch)]], rv0, s0)

        def body(i, carry):
            o0 = pl.multiple_of(2 * i * ch, 8)
            o1 = pl.multiple_of((2 * i + 1) * ch, 8)
            pltpu.async_copy(table_hbm.at[idx_v.at[pl.ds(o1, ch)]], rv1, s1)
            pltpu.make_async_copy(
                table_hbm.at[idx_v.at[pl.ds(o0, ch)]], rv0, s0
            ).wait()
            pltpu.sync_copy(rv0, out_hbm.at[pl.ds(base + o0, ch)])

            @pl.when(i + 1 < n_ch // 2)
            def _():
                o2 = pl.multiple_of((2 * i + 2) * ch, 8)
                pltpu.async_copy(
                    table_hbm.at[idx_v.at[pl.ds(o2, ch)]], rv0, s0
                )

            pltpu.make_async_copy(
                table_hbm.at[idx_v.at[pl.ds(o1, ch)]], rv1, s1
            ).wait()
            pltpu.sync_copy(rv1, out_hbm.at[pl.ds(base + o1, ch)])
            return carry

        lax.fori_loop(0, n_ch // 2, body, 0)

    return k


def _sc_gather(table, idx):
    n, d = table.shape
    (b,) = idx.shape
    return _make_sc_gather(d, b, table.dtype)(table, idx)


def _global_idx(ni, vcount):
    bs = ni.shape[0]
    off = (jnp.arange(bs, dtype=jnp.int32) * vcount)[:, None, None]
    return (ni.astype(jnp.int32) + off).reshape(-1)


def _gather_nbr(t, idx):
    bs, v, c = t.shape
    _, r, n = idx.shape
    g = _sc_gather(t.reshape(bs * v, c), _global_idx(idx, v))
    return g.reshape(bs, r, n, c)


def _gather_vert(vtx, idx):
    bs, v, _ = vtx.shape
    vp = jnp.pad(vtx, ((0, 0), (0, 0), (0, 13)))
    g = _sc_gather(vp.reshape(bs * v, 16), _global_idx(idx, v))
    return g.reshape(bs, v, idx.shape[2], 16)[..., :3]


_CAP = 64


def _make_knn_prep(bs, V, RB):
    nslots = V // 128

    def body(vt_ref, vtf_ref, d_ref, tau_ref):
        vrow = vt_ref[0]
        vall = vtf_ref[0]
        inner = lax.dot_general(vrow, vall, (((0,), (0,)), ((), ())))
        q = jnp.sum(vall * vall, axis=0)
        qr = jnp.sum(vrow * vrow, axis=0)
        d = qr[:, None] + q[None, :] - 2.0 * inner
        rblk = pl.program_id(1)
        rowg = rblk * RB + lax.broadcasted_iota(jnp.int32, (RB, V), 0)
        colg = lax.broadcasted_iota(jnp.int32, (RB, V), 1)
        d = jnp.where(rowg == colg, jnp.inf, d)
        d_ref[...] = d
        fold = jnp.full((RB, 128), jnp.inf, jnp.float32)
        for s in range(nslots):
            fold = jnp.minimum(fold, d[:, s * 128:(s + 1) * 128])
        lane = lax.broadcasted_iota(jnp.int32, (RB, 128), 1)
        tau = None
        for _ in range(20):
            m = jnp.min(fold, axis=1)
            cand = jnp.where(fold == m[:, None], lane, 1 << 30)
            aml = jnp.min(cand, axis=1)
            fold = jnp.where(lane == aml[:, None], jnp.inf, fold)
            tau = m
        tau_ref[0] = jnp.broadcast_to(tau[:, None], (RB, 16))

    nrb = V // RB
    return pl.pallas_call(
        body,
        grid=(bs, nrb),
        in_specs=[
            pl.BlockSpec((1, 3, RB), lambda b, r: (b, 0, r)),
            pl.BlockSpec((1, 3, V), lambda b, r: (b, 0, 0)),
        ],
        out_specs=[
            pl.BlockSpec((RB, V), lambda b, r: (b * nrb + r, 0)),
            pl.BlockSpec((1, RB, 16), lambda b, r: (b * nrb + r, 0, 0)),
        ],
        out_shape=[
            jax.ShapeDtypeStruct((bs * V, V), jnp.float32),
            jax.ShapeDtypeStruct((bs * nrb, RB, 16), jnp.float32),
        ],
    )


def _make_sc_filter(R, V):
    rpt = R // _NW
    assert rpt % 16 == 0
    n_ch = V // 16
    mesh = plsc.VectorSubcoreMesh(core_axis_name="c", subcore_axis_name="s")
    inf16 = float('inf')

    @functools.partial(
        pl.kernel,
        mesh=mesh,
        compiler_params=pltpu.CompilerParams(
            use_tc_tiling_on_sc=False, needs_layout_passes=False
        ),
        out_type=[
            jax.ShapeDtypeStruct((R * _CAP,), jnp.float32),
            jax.ShapeDtypeStruct((R * _CAP,), jnp.int32),
        ],
        scratch_types=[
            pltpu.VMEM((1, V), jnp.float32),
            pltpu.VMEM((1, V), jnp.float32),
            pltpu.VMEM((rpt * 16,), jnp.float32),
            pltpu.VMEM((16 * _CAP,), jnp.float32),
            pltpu.VMEM((16 * _CAP,), jnp.int32),
            pltpu.SemaphoreType.DMA,
            pltpu.SemaphoreType.DMA,
        ],
    )
    def k(d_hbm, tau_hbm, val_hbm, idx_hbm, db0, db1, taub, vbuf, ibuf, s0, s1):
        wid = lax.axis_index("s") * 2 + lax.axis_index("c")
        base = pl.multiple_of(wid * rpt, 8)
        pltpu.sync_copy(tau_hbm.at[pl.ds(base * 16, rpt * 16)], taub)

        def prefill():
            big = jnp.full((16,), jnp.inf, jnp.float32)
            for j in range(_CAP):
                vbuf[pl.ds(j * 16, 16)] = big

        def process(dref, r):
            gi = lax.rem(r, 16)
            li = lax.iota(jnp.int32, 16)
            tauv = taub[pl.ds(pl.multiple_of(r * 16, 8), 16)]
            gb = jnp.full((16,), gi * _CAP, jnp.int32)

            def chunk(c, n):
                dv = dref[0, pl.ds(pl.multiple_of(c * 16, 8), 16)]
                mask = dv <= tauv
                cum = plsc.cumsum(jnp.where(mask, 1, 0))
                pos = jnp.clip(n + cum - 1, 0, _CAP - 1)
                colv = li + c * 16
                plsc.store_scatter(vbuf, [gb + pos], dv, mask=mask)
                plsc.store_scatter(ibuf, [gb + pos], colv, mask=mask)
                return n + plsc.all_reduce_population_count(mask)

            lax.fori_loop(0, n_ch, chunk, jnp.zeros((16,), jnp.int32))

            @pl.when(gi == 15)
            def _():
                off = pl.multiple_of((base + r - 15) * _CAP, 8)
                pltpu.sync_copy(vbuf, val_hbm.at[pl.ds(off, 16 * _CAP)])
                pltpu.sync_copy(ibuf, idx_hbm.at[pl.ds(off, 16 * _CAP)])
                prefill()

        prefill()
        pltpu.async_copy(d_hbm.at[pl.ds(base, 1)], db0, s0)

        def outer(i, carry):
            r = 2 * i
            pltpu.async_copy(d_hbm.at[pl.ds(base + r + 1, 1)], db1, s1)
            pltpu.make_async_copy(d_hbm.at[pl.ds(base + r, 1)], db0, s0).wait()
            process(db0, r)

            @pl.when(i + 1 < rpt // 2)
            def _():
                pltpu.async_copy(d_hbm.at[pl.ds(base + r + 2, 1)], db0, s0)

            pltpu.make_async_copy(
                d_hbm.at[pl.ds(base + r + 1, 1)], db1, s1
            ).wait()
            process(db1, r + 1)
            return carry

        lax.fori_loop(0, rpt // 2, outer, 0)

    return k


def _make_knn_extract(R, RB):
    def body(val_ref, idx_ref, out_ref):
        v = val_ref[...]
        ii = idx_ref[...]
        lane = lax.broadcasted_iota(jnp.int32, (RB, _CAP), 1)
        for kk in range(20):
            m = jnp.min(v, axis=1)
            cand = jnp.where(v == m[:, None], lane, 1 << 30)
            aml = jnp.min(cand, axis=1)
            oh = lane == aml[:, None]
            nik = jnp.sum(jnp.where(oh, ii, 0), axis=1)
            out_ref[:, kk] = nik
            v = jnp.where(oh, jnp.inf, v)

    return pl.pallas_call(
        body,
        grid=(R // RB,),
        in_specs=[
            pl.BlockSpec((RB, _CAP), lambda r: (r, 0)),
            pl.BlockSpec((RB, _CAP), lambda r: (r, 0)),
        ],
        out_specs=pl.BlockSpec((RB, 32), lambda r: (r, 0)),
        out_shape=jax.ShapeDtypeStruct((R, 32), jnp.int32),
    )


def _knn_pallas(v):
    bs, V, _ = v.shape
    R = bs * V
    vt = jnp.transpose(v, (0, 2, 1))
    d, tau = _make_knn_prep(bs, V, min(256, V))(vt, vt)
    cval, cidx = _make_sc_filter(R, V)(d, tau.reshape(R * 16))
    ni = _make_knn_extract(R, 512)(
        cval.reshape(R, _CAP), cidx.reshape(R, _CAP)
    )
    return ni.reshape(bs, V, 32)[:, :, :20]


def _normalize(x, axis):
    norm = jnp.linalg.norm(x, axis=axis, keepdims=True)
    return x / jnp.maximum(norm, 1e-12)


def _knn(v, k):
    del k
    return _knn_pallas(v)


def _ndn(v, ni):
    nb = _gather_vert(v, ni)
    return _normalize(nb - v[:, :, None, :], -1)


def _conv_surface(ndn, dirs):
    sdn = _normalize(dirs, 0)
    theta = jax.nn.relu(ndn @ sdn)
    return jnp.max(theta, axis=2)


def _conv_layer(ni, ndn, fm, w, b, dirs, oc):
    sdn = _normalize(dirs, 0)
    theta = jax.nn.relu(ndn @ sdn)
    fo = fm @ w + b
    center = fo[:, :, :oc]
    support = _gather_nbr(fo[:, :, oc:], ni)
    act = jnp.max(theta * support, axis=2)
    return center + act


def _pool(v, fm, ni, rate):
    bs, vn, _ = v.shape
    samp = jnp.arange(vn // rate) * rate
    nf = _gather_nbr(fm, ni[:, samp, :])
    pooled = jnp.max(nf, axis=2)
    return v[:, samp, :], pooled


def kernel(vertices, dirs0, w1, b1, dirs1, w2, b2, dirs2, w3, b3, dirs3, w4, b4, dirs4):
    bs, _, vn, _ = vertices.shape
    v = vertices.reshape(bs, vn, 3)
    ni = _knn(v, _K)
    ndn = _ndn(v, ni)
    fm0 = jax.nn.relu(_conv_surface(ndn, dirs0))
    fm1 = jax.nn.relu(_conv_layer(ni, ndn, fm0, w1, b1, dirs1, 64))
    v, fm1 = _pool(v, fm1, ni, 4)
    ni = _knn(v, _K)
    ndn = _ndn(v, ni)
    fm2 = jax.nn.relu(_conv_layer(ni, ndn, fm1, w2, b2, dirs2, 128))
    fm3 = jax.nn.relu(_conv_layer(ni, ndn, fm2, w3, b3, dirs3, 256))
    v, fm3 = _pool(v, fm3, ni, 4)
    ni = _knn(v, _K)
    ndn = _ndn(v, ni)
    fm4 = _conv_layer(ni, ndn, fm3, w4, b4, dirs4, 1024)
    fm4 = jnp.transpose(fm4, (0, 2, 1))[..., None]
    return fm4


# speedup vs baseline: 2.6017x; 1.1614x over previous
import functools

import jax
import jax.numpy as jnp
from jax import lax
from jax.experimental import pallas as pl
from jax.experimental.pallas import tpu as pltpu
from jax.experimental.pallas import tpu_sc as plsc

_K = 20
_NW = 32


def _make_sc_gather(D, B, dtype):
    assert B % (8 * _NW) == 0 and D % 8 == 0
    b_per_w = B // _NW
    ch = 128
    while ch * D * 4 > 130 * 1024 and ch > 8:
        ch //= 2
    while b_per_w % (2 * ch):
        ch //= 2
    n_ch = b_per_w // ch
    assert n_ch % 2 == 0
    mesh = plsc.VectorSubcoreMesh(core_axis_name="c", subcore_axis_name="s")

    @functools.partial(
        pl.kernel,
        mesh=mesh,
        compiler_params=pltpu.CompilerParams(use_tc_tiling_on_sc=False),
        out_type=jax.ShapeDtypeStruct((B, D), dtype),
        scratch_types=[
            pltpu.VMEM((b_per_w,), jnp.int32),
            pltpu.VMEM((ch, D), dtype),
            pltpu.VMEM((ch, D), dtype),
            pltpu.SemaphoreType.DMA,
            pltpu.SemaphoreType.DMA,
        ],
    )
    def k(table_hbm, idx_hbm, out_hbm, idx_v, rv0, rv1, s0, s1):
        wid = lax.axis_index("s") * 2 + lax.axis_index("c")
        base = pl.multiple_of(wid * b_per_w, 8)
        pltpu.sync_copy(idx_hbm.at[pl.ds(base, b_per_w)], idx_v)
        pltpu.async_copy(table_hbm.at[idx_v.at[pl.ds(0, ch)]], rv0, s0)

        def body(i, carry):
            o0 = pl.multiple_of(2 * i * ch, 8)
            o1 = pl.multiple_of((2 * i + 1) * ch, 8)
            pltpu.async_copy(table_hbm.at[idx_v.at[pl.ds(o1, ch)]], rv1, s1)
            pltpu.make_async_copy(
                table_hbm.at[idx_v.at[pl.ds(o0, ch)]], rv0, s0
            ).wait()
            pltpu.sync_copy(rv0, out_hbm.at[pl.ds(base + o0, ch)])

            @pl.when(i + 1 < n_ch // 2)
            def _():
                o2 = pl.multiple_of((2 * i + 2) * ch, 8)
                pltpu.async_copy(
                    table_hbm.at[idx_v.at[pl.ds(o2, ch)]], rv0, s0
                )

            pltpu.make_async_copy(
                table_hbm.at[idx_v.at[pl.ds(o1, ch)]], rv1, s1
            ).wait()
            pltpu.sync_copy(rv1, out_hbm.at[pl.ds(base + o1, ch)])
            return carry

        lax.fori_loop(0, n_ch // 2, body, 0)

    return k


def _sc_gather(table, idx):
    n, d = table.shape
    (b,) = idx.shape
    return _make_sc_gather(d, b, table.dtype)(table, idx)


def _global_idx(ni, vcount):
    bs = ni.shape[0]
    off = (jnp.arange(bs, dtype=jnp.int32) * vcount)[:, None, None]
    return (ni.astype(jnp.int32) + off).reshape(-1)


def _gather_nbr(t, idx):
    bs, v, c = t.shape
    _, r, n = idx.shape
    g = _sc_gather(t.reshape(bs * v, c), _global_idx(idx, v))
    return g.reshape(bs, r, n, c)


def _make_sc_plane_gather(N, B):
    b_per_w = B // _NW
    ch = 2048
    while b_per_w % ch or ch % 16:
        ch //= 2
    n_ch = b_per_w // ch
    mesh = plsc.VectorSubcoreMesh(core_axis_name="c", subcore_axis_name="s")

    @functools.partial(
        pl.kernel,
        mesh=mesh,
        compiler_params=pltpu.CompilerParams(
            use_tc_tiling_on_sc=False, needs_layout_passes=False
        ),
        out_type=[jax.ShapeDtypeStruct((B,), jnp.float32)] * 3,
        scratch_types=[
            pltpu.VMEM((N,), jnp.float32),
            pltpu.VMEM((N,), jnp.float32),
            pltpu.VMEM((N,), jnp.float32),
            pltpu.VMEM((ch,), jnp.int32),
            pltpu.VMEM((ch,), jnp.float32),
            pltpu.VMEM((ch,), jnp.float32),
            pltpu.VMEM((ch,), jnp.float32),
        ],
    )
    def k(xh, yh, zh, ih, oxh, oyh, ozh, xt, yt, zt, ib, ox, oy, oz):
        wid = lax.axis_index("s") * 2 + lax.axis_index("c")
        base = pl.multiple_of(wid * b_per_w, 8)
        pltpu.sync_copy(xh, xt)
        pltpu.sync_copy(yh, yt)
        pltpu.sync_copy(zh, zt)

        def body(c, carry):
            off = pl.multiple_of(base + c * ch, 8)
            pltpu.sync_copy(ih.at[pl.ds(off, ch)], ib)
            for j in range(ch // 16):
                iv = ib[pl.ds(j * 16, 16)]
                ox[pl.ds(j * 16, 16)] = plsc.load_gather(xt, [iv])
                oy[pl.ds(j * 16, 16)] = plsc.load_gather(yt, [iv])
                oz[pl.ds(j * 16, 16)] = plsc.load_gather(zt, [iv])
            pltpu.sync_copy(ox, oxh.at[pl.ds(off, ch)])
            pltpu.sync_copy(oy, oyh.at[pl.ds(off, ch)])
            pltpu.sync_copy(oz, ozh.at[pl.ds(off, ch)])
            return carry

        lax.fori_loop(0, n_ch, body, 0)

    return k


def _gather_vert(vt, idx):
    bs, _, v = vt.shape
    n = idx.shape[2]
    gi = _global_idx(idx, v)
    xp = vt[:, 0, :].reshape(bs * v)
    yp = vt[:, 1, :].reshape(bs * v)
    zp = vt[:, 2, :].reshape(bs * v)
    gx, gy, gz = _make_sc_plane_gather(bs * v, gi.shape[0])(xp, yp, zp, gi)
    return jnp.stack([gx, gy, gz], axis=-1).reshape(bs, v, n, 3)


_CAP = 64


def _make_knn_prep(bs, V, RB):
    nslots = V // 128

    def body(vt_ref, vtf_ref, d_ref, tau_ref):
        vrow = vt_ref[0]
        vall = vtf_ref[0]
        inner = lax.dot_general(vrow, vall, (((0,), (0,)), ((), ())))
        q = jnp.sum(vall * vall, axis=0)
        qr = jnp.sum(vrow * vrow, axis=0)
        d = qr[:, None] + q[None, :] - 2.0 * inner
        rblk = pl.program_id(1)
        rowg = rblk * RB + lax.broadcasted_iota(jnp.int32, (RB, V), 0)
        colg = lax.broadcasted_iota(jnp.int32, (RB, V), 1)
        d = jnp.where(rowg == colg, jnp.inf, d)
        d_ref[...] = d
        fold = jnp.full((RB, 128), jnp.inf, jnp.float32)
        for s in range(nslots):
            fold = jnp.minimum(fold, d[:, s * 128:(s + 1) * 128])
        lane = lax.broadcasted_iota(jnp.int32, (RB, 128), 1)
        tau = None
        for _ in range(20):
            m = jnp.min(fold, axis=1)
            cand = jnp.where(fold == m[:, None], lane, 1 << 30)
            aml = jnp.min(cand, axis=1)
            fold = jnp.where(lane == aml[:, None], jnp.inf, fold)
            tau = m
        tau_ref[0] = jnp.broadcast_to(tau[:, None], (RB, 16))

    nrb = V // RB
    return pl.pallas_call(
        body,
        grid=(bs, nrb),
        in_specs=[
            pl.BlockSpec((1, 3, RB), lambda b, r: (b, 0, r)),
            pl.BlockSpec((1, 3, V), lambda b, r: (b, 0, 0)),
        ],
        out_specs=[
            pl.BlockSpec((RB, V), lambda b, r: (b * nrb + r, 0)),
            pl.BlockSpec((1, RB, 16), lambda b, r: (b * nrb + r, 0, 0)),
        ],
        out_shape=[
            jax.ShapeDtypeStruct((bs * V, V), jnp.float32),
            jax.ShapeDtypeStruct((bs * nrb, RB, 16), jnp.float32),
        ],
    )


def _make_sc_filter(R, V):
    rpt = R // _NW
    assert rpt % 16 == 0
    n_ch = V // 16
    mesh = plsc.VectorSubcoreMesh(core_axis_name="c", subcore_axis_name="s")
    inf16 = float('inf')

    @functools.partial(
        pl.kernel,
        mesh=mesh,
        compiler_params=pltpu.CompilerParams(
            use_tc_tiling_on_sc=False, needs_layout_passes=False
        ),
        out_type=[
            jax.ShapeDtypeStruct((R * _CAP,), jnp.float32),
            jax.ShapeDtypeStruct((R * _CAP,), jnp.int32),
        ],
        scratch_types=[
            pltpu.VMEM((1, V), jnp.float32),
            pltpu.VMEM((1, V), jnp.float32),
            pltpu.VMEM((rpt * 16,), jnp.float32),
            pltpu.VMEM((16 * _CAP,), jnp.float32),
            pltpu.VMEM((16 * _CAP,), jnp.int32),
            pltpu.SemaphoreType.DMA,
            pltpu.SemaphoreType.DMA,
        ],
    )
    def k(d_hbm, tau_hbm, val_hbm, idx_hbm, db0, db1, taub, vbuf, ibuf, s0, s1):
        wid = lax.axis_index("s") * 2 + lax.axis_index("c")
        base = pl.multiple_of(wid * rpt, 8)
        pltpu.sync_copy(tau_hbm.at[pl.ds(base * 16, rpt * 16)], taub)

        def prefill():
            big = jnp.full((16,), jnp.inf, jnp.float32)
            for j in range(_CAP):
                vbuf[pl.ds(j * 16, 16)] = big

        def process(dref, r):
            gi = lax.rem(r, 16)
            li = lax.iota(jnp.int32, 16)
            tauv = taub[pl.ds(pl.multiple_of(r * 16, 8), 16)]
            gb = jnp.full((16,), gi * _CAP, jnp.int32)

            def chunk(c, n):
                dv = dref[0, pl.ds(pl.multiple_of(c * 16, 8), 16)]
                mask = dv <= tauv
                cum = plsc.cumsum(jnp.where(mask, 1, 0))
                pos = jnp.clip(n + cum - 1, 0, _CAP - 1)
                colv = li + c * 16
                plsc.store_scatter(vbuf, [gb + pos], dv, mask=mask)
                plsc.store_scatter(ibuf, [gb + pos], colv, mask=mask)
                return n + plsc.all_reduce_population_count(mask)

            lax.fori_loop(0, n_ch, chunk, jnp.zeros((16,), jnp.int32))

            @pl.when(gi == 15)
            def _():
                off = pl.multiple_of((base + r - 15) * _CAP, 8)
                pltpu.sync_copy(vbuf, val_hbm.at[pl.ds(off, 16 * _CAP)])
                pltpu.sync_copy(ibuf, idx_hbm.at[pl.ds(off, 16 * _CAP)])
                prefill()

        prefill()
        pltpu.async_copy(d_hbm.at[pl.ds(base, 1)], db0, s0)

        def outer(i, carry):
            r = 2 * i
            pltpu.async_copy(d_hbm.at[pl.ds(base + r + 1, 1)], db1, s1)
            pltpu.make_async_copy(d_hbm.at[pl.ds(base + r, 1)], db0, s0).wait()
            process(db0, r)

            @pl.when(i + 1 < rpt // 2)
            def _():
                pltpu.async_copy(d_hbm.at[pl.ds(base + r + 2, 1)], db0, s0)

            pltpu.make_async_copy(
                d_hbm.at[pl.ds(base + r + 1, 1)], db1, s1
            ).wait()
            process(db1, r + 1)
            return carry

        lax.fori_loop(0, rpt // 2, outer, 0)

    return k


def _make_knn_extract(R, RB):
    def body(val_ref, idx_ref, out_ref):
        v = val_ref[...]
        ii = idx_ref[...]
        lane = lax.broadcasted_iota(jnp.int32, (RB, _CAP), 1)
        for kk in range(20):
            m = jnp.min(v, axis=1)
            cand = jnp.where(v == m[:, None], lane, 1 << 30)
            aml = jnp.min(cand, axis=1)
            oh = lane == aml[:, None]
            nik = jnp.sum(jnp.where(oh, ii, 0), axis=1)
            out_ref[:, kk] = nik
            v = jnp.where(oh, jnp.inf, v)

    return pl.pallas_call(
        body,
        grid=(R // RB,),
        in_specs=[
            pl.BlockSpec((RB, _CAP), lambda r: (r, 0)),
            pl.BlockSpec((RB, _CAP), lambda r: (r, 0)),
        ],
        out_specs=pl.BlockSpec((RB, 32), lambda r: (r, 0)),
        out_shape=jax.ShapeDtypeStruct((R, 32), jnp.int32),
    )


def _knn_pallas(v):
    bs, V, _ = v.shape
    R = bs * V
    vt = jnp.transpose(v, (0, 2, 1))
    d, tau = _make_knn_prep(bs, V, min(256, V))(vt, vt)
    cval, cidx = _make_sc_filter(R, V)(d, tau.reshape(R * 16))
    ni = _make_knn_extract(R, 512)(
        cval.reshape(R, _CAP), cidx.reshape(R, _CAP)
    )
    return ni.reshape(bs, V, 32)[:, :, :20]


def _normalize(x, axis):
    norm = jnp.linalg.norm(x, axis=axis, keepdims=True)
    return x / jnp.maximum(norm, 1e-12)


def _knn(v, k):
    del k
    return _knn_pallas(v)


def _ndn(v, ni):
    vt = jnp.transpose(v, (0, 2, 1))
    nb = _gather_vert(vt, ni)
    return _normalize(nb - v[:, :, None, :], -1)


def _conv_surface(ndn, dirs):
    sdn = _normalize(dirs, 0)
    theta = jax.nn.relu(ndn @ sdn)
    return jnp.max(theta, axis=2)


def _conv_layer(ni, ndn, fm, w, b, dirs, oc):
    sdn = _normalize(dirs, 0)
    theta = jax.nn.relu(ndn @ sdn)
    fo = fm @ w + b
    center = fo[:, :, :oc]
    support = _gather_nbr(fo[:, :, oc:], ni)
    act = jnp.max(theta * support, axis=2)
    return center + act


def _pool(v, fm, ni, rate):
    bs, vn, _ = v.shape
    samp = jnp.arange(vn // rate) * rate
    nf = _gather_nbr(fm, ni[:, samp, :])
    pooled = jnp.max(nf, axis=2)
    return v[:, samp, :], pooled


def kernel(vertices, dirs0, w1, b1, dirs1, w2, b2, dirs2, w3, b3, dirs3, w4, b4, dirs4):
    bs, _, vn, _ = vertices.shape
    v = vertices.reshape(bs, vn, 3)
    ni = _knn(v, _K)
    ndn = _ndn(v, ni)
    fm0 = jax.nn.relu(_conv_surface(ndn, dirs0))
    fm1 = jax.nn.relu(_conv_layer(ni, ndn, fm0, w1, b1, dirs1, 64))
    v, fm1 = _pool(v, fm1, ni, 4)
    ni = _knn(v, _K)
    ndn = _ndn(v, ni)
    fm2 = jax.nn.relu(_conv_layer(ni, ndn, fm1, w2, b2, dirs2, 128))
    fm3 = jax.nn.relu(_conv_layer(ni, ndn, fm2, w3, b3, dirs3, 256))
    v, fm3 = _pool(v, fm3, ni, 4)
    ni = _knn(v, _K)
    ndn = _ndn(v, ni)
    fm4 = _conv_layer(ni, ndn, fm3, w4, b4, dirs4, 1024)
    fm4 = jnp.transpose(fm4, (0, 2, 1))[..., None]
    return fm4
